# baseline (speedup 1.0000x reference)
import functools

import jax
import jax.numpy as jnp
from jax import lax
from jax.experimental import pallas as pl
from jax.experimental.pallas import tpu as pltpu
from jax.experimental.pallas import tpu_sc as plsc

B = 4096
T = 200
D = 32
V = 1000000
NC = 2
NS = 16
NW = NC * NS
G = 2
NG = T // G
GR = G * 128
WROWS = G * 4
U = 8
TI = 512
NBLK = (V + TI - 1) // TI
PR = NBLK * 128


def _tc_detile(tt):
    def body(in_ref, out_ref):
        blk = in_ref[...]
        cols = [jnp.transpose(blk[:, 128 * a:128 * (a + 1)])
                for a in range(4)]
        out_ref[...] = jnp.concatenate(cols, axis=1)

    return pl.pallas_call(
        body,
        grid=(NBLK,),
        in_specs=[pl.BlockSpec((D, TI), lambda i: (0, i))],
        out_specs=pl.BlockSpec((128, 128), lambda i: (i, 0)),
        out_shape=jax.ShapeDtypeStruct((PR, 128), jnp.float32),
    )(tt)


def _sc_embed(xtf, packed, pos_table):
    mesh = plsc.VectorSubcoreMesh(core_axis_name="c", subcore_axis_name="s")

    @functools.partial(
        pl.kernel,
        mesh=mesh,
        out_type=jax.ShapeDtypeStruct((T * 4, NW, 8 * 128), jnp.float32),
        scratch_types=[
            pltpu.VMEM((2, GR), jnp.int32),
            pltpu.VMEM((2, GR), jnp.int32),
            pltpu.VMEM((2, GR), jnp.int32),
            pltpu.VMEM((2, GR, 128), jnp.float32),
            pltpu.VMEM((2, WROWS, 1024), jnp.float32),
            pltpu.VMEM((T, D), jnp.float32),
            pltpu.SemaphoreType.DMA((2,)),
            pltpu.SemaphoreType.DMA((2,)),
            pltpu.SemaphoreType.DMA((2,)),
        ],
        compiler_params=pltpu.CompilerParams(use_tc_tiling_on_sc=False,
                                             needs_layout_passes=False,
                                             disable_bounds_checks=True),
    )
    def body(x_hbm, tok_hbm, pos_hbm, out_hbm,
             idx_v, row_v, off_v, rows_v, wout_v, pos_v, isem, gsem, wsem):
        wid = lax.axis_index("s") * NC + lax.axis_index("c")
        col0 = wid * 128
        pltpu.sync_copy(pos_hbm, pos_v)

        io = lax.iota(jnp.int32, 16)
        rowp = lax.shift_right_logical(io, 3)
        colp = (io & 7) * 128

        def idx_start(g, b):
            t0 = g * G
            for k in range(G):
                pltpu.async_copy(x_hbm.at[pl.ds((t0 + k) * B + col0, 128)],
                                 idx_v.at[b, pl.ds(k * 128, 128)], isem.at[b])

        def idx_wait(b):
            for k in range(G):
                pltpu.make_async_copy(
                    x_hbm.at[pl.ds(col0, 128)],
                    idx_v.at[b, pl.ds(k * 128, 128)], isem.at[b]).wait()

        def idx_transform(b):
            for i in range(GR // 16):
                v = idx_v[b, pl.ds(i * 16, 16)]
                row = (lax.shift_right_logical(v, 9) * 128) | (v & 127)
                off = (lax.shift_right_logical(v, 7) & 3) * 32
                row_v[b, pl.ds(i * 16, 16)] = row
                off_v[b, pl.ds(i * 16, 16)] = off

        def gather_start(b):
            pltpu.async_copy(tok_hbm.at[row_v.at[b]], rows_v.at[b],
                             gsem.at[b])

        def gather_wait(b):
            pltpu.make_async_copy(tok_hbm.at[row_v.at[b]], rows_v.at[b],
                                  gsem.at[b]).wait()

        def write_start(g, b):
            t0 = g * G
            pltpu.async_copy(wout_v.at[b],
                             out_hbm.at[pl.ds(t0 * 4, WROWS), wid],
                             wsem.at[b])

        def write_wait(b):
            pltpu.make_async_copy(wout_v.at[b],
                                  out_hbm.at[pl.ds(0, WROWS), wid],
                                  wsem.at[b]).wait()

        def compute(g, b):
            t0 = g * G
            for k in range(G):
                pv0 = pos_v[t0 + k, pl.ds(0, 16)]
                pv1 = pos_v[t0 + k, pl.ds(16, 16)]
                rv0 = rowp + (k * 4)
                rv1 = rv0 + 2

                @plsc.parallel_loop(0, 128, step=1, unroll=U)
                def _(bl):
                    tp = k * 128 + bl
                    rsplat = jnp.full((16,), tp, jnp.int32)
                    offs = plsc.load_gather(off_v.at[b], [rsplat])
                    c0 = offs + io
                    cv = colp + bl
                    plsc.store_scatter(
                        wout_v.at[b], [rv0, cv],
                        plsc.load_gather(rows_v.at[b], [rsplat, c0]) + pv0)
                    plsc.store_scatter(
                        wout_v.at[b], [rv1, cv],
                        plsc.load_gather(rows_v.at[b], [rsplat, c0 + 16])
                        + pv1)

        idx_start(0, 0)
        idx_start(1, 1)
        idx_wait(0)
        idx_transform(0)
        gather_start(0)

        def ring(i, c):
            for b in (0, 1):
                g = 2 * i + b
                nb = 1 - b

                @pl.when(g + 1 < NG)
                def _():
                    idx_wait(nb)
                    idx_transform(nb)
                    gather_start(nb)

                gather_wait(b)

                @pl.when(g + 2 < NG)
                def _():
                    idx_start(g + 2, b)

                @pl.when(g >= 2)
                def _():
                    write_wait(b)

                compute(g, b)
                write_start(g, b)
            return c

        lax.fori_loop(0, NG // 2, ring, 0)
        write_wait(0)
        write_wait(1)

    return body(xtf, packed, pos_table)


def kernel(x, token_table, pos_table):
    xtf = x.T.reshape(B * T)
    packed = _tc_detile(token_table.T)
    out = _sc_embed(xtf, packed, pos_table)
    return (out.reshape(T, 4, NW, 8, 128).transpose(2, 4, 0, 1, 3)
            .reshape(B, T, D))


# speedup vs baseline: 1.8825x; 1.8825x over previous
import functools

import jax
import jax.numpy as jnp
from jax import lax
from jax.experimental import pallas as pl
from jax.experimental.pallas import tpu as pltpu
from jax.experimental.pallas import tpu_sc as plsc

B = 4096
T = 200
D = 32
V = 1000000
NC = 2
NS = 16
NW = NC * NS
G = 2
NG = T // G
GR = G * 128
WROWS = G * 4
U = 8
TI = 2048
NBLK = (V + TI - 1) // TI
PR = NBLK * (TI // 4)


def _tc_detile(tt):
    def body(in_ref, out_ref):
        tr = jnp.transpose(in_ref[...])
        packs = [
            jnp.concatenate(
                [tr[512 * q + 128 * a:512 * q + 128 * (a + 1), :]
                 for a in range(4)], axis=1)
            for q in range(TI // 512)]
        out_ref[...] = jnp.concatenate(packs, axis=0)

    return pl.pallas_call(
        body,
        grid=(NBLK,),
        in_specs=[pl.BlockSpec((D, TI), lambda i: (0, i))],
        out_specs=pl.BlockSpec((TI // 4, 128), lambda i: (i, 0)),
        out_shape=jax.ShapeDtypeStruct((PR, 128), jnp.float32),
    )(tt)


def _sc_embed(xtf, packed, pos_table):
    mesh = plsc.VectorSubcoreMesh(core_axis_name="c", subcore_axis_name="s")

    @functools.partial(
        pl.kernel,
        mesh=mesh,
        out_type=jax.ShapeDtypeStruct((T * 4, NW, 8 * 128), jnp.float32),
        scratch_types=[
            pltpu.VMEM((2, GR), jnp.int32),
            pltpu.VMEM((2, GR), jnp.int32),
            pltpu.VMEM((2, GR), jnp.int32),
            pltpu.VMEM((2, GR, 128), jnp.float32),
            pltpu.VMEM((2, WROWS, 1024), jnp.float32),
            pltpu.VMEM((T, D), jnp.float32),
            pltpu.SemaphoreType.DMA((2,)),
            pltpu.SemaphoreType.DMA((2,)),
            pltpu.SemaphoreType.DMA((2,)),
        ],
        compiler_params=pltpu.CompilerParams(use_tc_tiling_on_sc=False,
                                             needs_layout_passes=False,
                                             disable_bounds_checks=True),
    )
    def body(x_hbm, tok_hbm, pos_hbm, out_hbm,
             idx_v, row_v, off_v, rows_v, wout_v, pos_v, isem, gsem, wsem):
        wid = lax.axis_index("s") * NC + lax.axis_index("c")
        col0 = wid * 128
        pltpu.sync_copy(pos_hbm, pos_v)

        io = lax.iota(jnp.int32, 16)
        rowp = lax.shift_right_logical(io, 3)
        colp = (io & 7) * 128

        def idx_start(g, b):
            t0 = g * G
            for k in range(G):
                pltpu.async_copy(x_hbm.at[pl.ds((t0 + k) * B + col0, 128)],
                                 idx_v.at[b, pl.ds(k * 128, 128)], isem.at[b])

        def idx_wait(b):
            for k in range(G):
                pltpu.make_async_copy(
                    x_hbm.at[pl.ds(col0, 128)],
                    idx_v.at[b, pl.ds(k * 128, 128)], isem.at[b]).wait()

        def idx_transform(b):
            for i in range(GR // 16):
                v = idx_v[b, pl.ds(i * 16, 16)]
                row = (lax.shift_right_logical(v, 9) * 128) | (v & 127)
                off = (lax.shift_right_logical(v, 7) & 3) * 32
                row_v[b, pl.ds(i * 16, 16)] = row
                off_v[b, pl.ds(i * 16, 16)] = off

        def gather_start(b):
            pltpu.async_copy(tok_hbm.at[row_v.at[b]], rows_v.at[b],
                             gsem.at[b])

        def gather_wait(b):
            pltpu.make_async_copy(tok_hbm.at[row_v.at[b]], rows_v.at[b],
                                  gsem.at[b]).wait()

        def write_start(g, b):
            t0 = g * G
            pltpu.async_copy(wout_v.at[b],
                             out_hbm.at[pl.ds(t0 * 4, WROWS), wid],
                             wsem.at[b])

        def write_wait(b):
            pltpu.make_async_copy(wout_v.at[b],
                                  out_hbm.at[pl.ds(0, WROWS), wid],
                                  wsem.at[b]).wait()

        def compute(g, b):
            t0 = g * G
            for k in range(G):
                pv0 = pos_v[t0 + k, pl.ds(0, 16)]
                pv1 = pos_v[t0 + k, pl.ds(16, 16)]
                rv0 = rowp + (k * 4)
                rv1 = rv0 + 2

                @plsc.parallel_loop(0, 128, step=1, unroll=U)
                def _(bl):
                    tp = k * 128 + bl
                    rsplat = jnp.full((16,), tp, jnp.int32)
                    offs = plsc.load_gather(off_v.at[b], [rsplat])
                    c0 = offs + io
                    cv = colp + bl
                    plsc.store_scatter(
                        wout_v.at[b], [rv0, cv],
                        plsc.load_gather(rows_v.at[b], [rsplat, c0]) + pv0)
                    plsc.store_scatter(
                        wout_v.at[b], [rv1, cv],
                        plsc.load_gather(rows_v.at[b], [rsplat, c0 + 16])
                        + pv1)

        idx_start(0, 0)
        idx_start(1, 1)
        idx_wait(0)
        idx_transform(0)
        gather_start(0)

        def ring(i, c):
            for b in (0, 1):
                g = 2 * i + b
                nb = 1 - b

                @pl.when(g + 1 < NG)
                def _():
                    idx_wait(nb)
                    idx_transform(nb)
                    gather_start(nb)

                gather_wait(b)

                @pl.when(g + 2 < NG)
                def _():
                    idx_start(g + 2, b)

                @pl.when(g >= 2)
                def _():
                    write_wait(b)

                compute(g, b)
                write_start(g, b)
            return c

        lax.fori_loop(0, NG // 2, ring, 0)
        write_wait(0)
        write_wait(1)

    return body(xtf, packed, pos_table)


def kernel(x, token_table, pos_table):
    xtf = x.T.reshape(B * T)
    packed = _tc_detile(token_table.T)
    out = _sc_embed(xtf, packed, pos_table)
    return (out.reshape(T, 4, NW, 8, 128).transpose(2, 4, 0, 1, 3)
            .reshape(B, T, D))


# speedup vs baseline: 2.3347x; 1.2402x over previous
import functools

import jax
import jax.numpy as jnp
from jax import lax
from jax.experimental import pallas as pl
from jax.experimental.pallas import tpu as pltpu
from jax.experimental.pallas import tpu_sc as plsc

B = 4096
T = 200
D = 32
V = 1000000
NC = 2
NS = 16
NW = NC * NS
G = 2
NG = T // G
GR = G * 128
WROWS = G * 4
U = 8
TI = 8192
NBLK = (V + TI - 1) // TI
PR = NBLK * (TI // 4)


def _tc_detile(tt):
    def body(in_ref, out_ref):
        tr = jnp.transpose(in_ref[...])
        packs = [
            jnp.concatenate(
                [tr[512 * q + 128 * a:512 * q + 128 * (a + 1), :]
                 for a in range(4)], axis=1)
            for q in range(TI // 512)]
        out_ref[...] = jnp.concatenate(packs, axis=0)

    return pl.pallas_call(
        body,
        grid=(NBLK,),
        in_specs=[pl.BlockSpec((D, TI), lambda i: (0, i))],
        out_specs=pl.BlockSpec((TI // 4, 128), lambda i: (i, 0)),
        out_shape=jax.ShapeDtypeStruct((PR, 128), jnp.float32),
    )(tt)


def _sc_embed(xtf, packed, pos_table):
    mesh = plsc.VectorSubcoreMesh(core_axis_name="c", subcore_axis_name="s")

    @functools.partial(
        pl.kernel,
        mesh=mesh,
        out_type=jax.ShapeDtypeStruct((T * 4, NW, 8 * 128), jnp.float32),
        scratch_types=[
            pltpu.VMEM((2, GR), jnp.int32),
            pltpu.VMEM((2, GR), jnp.int32),
            pltpu.VMEM((2, GR), jnp.int32),
            pltpu.VMEM((2, GR, 128), jnp.float32),
            pltpu.VMEM((2, WROWS, 1024), jnp.float32),
            pltpu.VMEM((T, D), jnp.float32),
            pltpu.SemaphoreType.DMA((2,)),
            pltpu.SemaphoreType.DMA((2,)),
            pltpu.SemaphoreType.DMA((2,)),
        ],
        compiler_params=pltpu.CompilerParams(use_tc_tiling_on_sc=False,
                                             needs_layout_passes=False,
                                             disable_bounds_checks=True),
    )
    def body(x_hbm, tok_hbm, pos_hbm, out_hbm,
             idx_v, row_v, off_v, rows_v, wout_v, pos_v, isem, gsem, wsem):
        wid = lax.axis_index("s") * NC + lax.axis_index("c")
        col0 = wid * 128
        pltpu.sync_copy(pos_hbm, pos_v)

        io = lax.iota(jnp.int32, 16)
        rowp = lax.shift_right_logical(io, 3)
        colp = (io & 7) * 128

        def idx_start(g, b):
            t0 = g * G
            for k in range(G):
                pltpu.async_copy(x_hbm.at[pl.ds((t0 + k) * B + col0, 128)],
                                 idx_v.at[b, pl.ds(k * 128, 128)], isem.at[b])

        def idx_wait(b):
            for k in range(G):
                pltpu.make_async_copy(
                    x_hbm.at[pl.ds(col0, 128)],
                    idx_v.at[b, pl.ds(k * 128, 128)], isem.at[b]).wait()

        def idx_transform(b):
            for i in range(GR // 16):
                v = idx_v[b, pl.ds(i * 16, 16)]
                row = (lax.shift_right_logical(v, 9) * 128) | (v & 127)
                off = (lax.shift_right_logical(v, 7) & 3) * 32
                row_v[b, pl.ds(i * 16, 16)] = row
                off_v[b, pl.ds(i * 16, 16)] = off

        def gather_start(b):
            pltpu.async_copy(tok_hbm.at[row_v.at[b]], rows_v.at[b],
                             gsem.at[b])

        def gather_wait(b):
            pltpu.make_async_copy(tok_hbm.at[row_v.at[b]], rows_v.at[b],
                                  gsem.at[b]).wait()

        def write_start(g, b):
            t0 = g * G
            pltpu.async_copy(wout_v.at[b],
                             out_hbm.at[pl.ds(t0 * 4, WROWS), wid],
                             wsem.at[b])

        def write_wait(b):
            pltpu.make_async_copy(wout_v.at[b],
                                  out_hbm.at[pl.ds(0, WROWS), wid],
                                  wsem.at[b]).wait()

        def compute(g, b):
            t0 = g * G
            for k in range(G):
                pv0 = pos_v[t0 + k, pl.ds(0, 16)]
                pv1 = pos_v[t0 + k, pl.ds(16, 16)]
                rv0 = rowp + (k * 4)
                rv1 = rv0 + 2

                @plsc.parallel_loop(0, 128, step=1, unroll=U)
                def _(bl):
                    tp = k * 128 + bl
                    rsplat = jnp.full((16,), tp, jnp.int32)
                    offs = plsc.load_gather(off_v.at[b], [rsplat])
                    c0 = offs + io
                    cv = colp + bl
                    plsc.store_scatter(
                        wout_v.at[b], [rv0, cv],
                        plsc.load_gather(rows_v.at[b], [rsplat, c0]) + pv0)
                    plsc.store_scatter(
                        wout_v.at[b], [rv1, cv],
                        plsc.load_gather(rows_v.at[b], [rsplat, c0 + 16])
                        + pv1)

        idx_start(0, 0)
        idx_start(1, 1)
        idx_wait(0)
        idx_transform(0)
        gather_start(0)

        def ring(i, c):
            for b in (0, 1):
                g = 2 * i + b
                nb = 1 - b

                @pl.when(g + 1 < NG)
                def _():
                    idx_wait(nb)
                    idx_transform(nb)
                    gather_start(nb)

                gather_wait(b)

                @pl.when(g + 2 < NG)
                def _():
                    idx_start(g + 2, b)

                @pl.when(g >= 2)
                def _():
                    write_wait(b)

                compute(g, b)
                write_start(g, b)
            return c

        lax.fori_loop(0, NG // 2, ring, 0)
        write_wait(0)
        write_wait(1)

    return body(xtf, packed, pos_table)


def kernel(x, token_table, pos_table):
    xtf = x.T.reshape(B * T)
    packed = _tc_detile(token_table.T)
    out = _sc_embed(xtf, packed, pos_table)
    return (out.reshape(T, 4, NW, 8, 128).transpose(2, 4, 0, 1, 3)
            .reshape(B, T, D))


# speedup vs baseline: 2.3545x; 1.0085x over previous
import functools

import jax
import jax.numpy as jnp
from jax import lax
from jax.experimental import pallas as pl
from jax.experimental.pallas import tpu as pltpu
from jax.experimental.pallas import tpu_sc as plsc

B = 4096
T = 200
D = 32
V = 1000000
NC = 2
NS = 16
NW = NC * NS
G = 2
NG = T // G
GR = G * 128
WROWS = G * 4
U = 8
TI = 32768
NBLK = (V + TI - 1) // TI
PR = NBLK * (TI // 4)


def _tc_detile(tt):
    def body(in_ref, out_ref):
        tr = jnp.transpose(in_ref[...])
        packs = [
            jnp.concatenate(
                [tr[512 * q + 128 * a:512 * q + 128 * (a + 1), :]
                 for a in range(4)], axis=1)
            for q in range(TI // 512)]
        out_ref[...] = jnp.concatenate(packs, axis=0)

    return pl.pallas_call(
        body,
        grid=(NBLK,),
        in_specs=[pl.BlockSpec((D, TI), lambda i: (0, i))],
        out_specs=pl.BlockSpec((TI // 4, 128), lambda i: (i, 0)),
        out_shape=jax.ShapeDtypeStruct((PR, 128), jnp.float32),
    )(tt)


def _sc_embed(xtf, packed, pos_table):
    mesh = plsc.VectorSubcoreMesh(core_axis_name="c", subcore_axis_name="s")

    @functools.partial(
        pl.kernel,
        mesh=mesh,
        out_type=jax.ShapeDtypeStruct((T * 4, NW, 8 * 128), jnp.float32),
        scratch_types=[
            pltpu.VMEM((2, GR), jnp.int32),
            pltpu.VMEM((2, GR), jnp.int32),
            pltpu.VMEM((2, GR), jnp.int32),
            pltpu.VMEM((2, GR, 128), jnp.float32),
            pltpu.VMEM((2, WROWS, 1024), jnp.float32),
            pltpu.VMEM((T, D), jnp.float32),
            pltpu.SemaphoreType.DMA((2,)),
            pltpu.SemaphoreType.DMA((2,)),
            pltpu.SemaphoreType.DMA((2,)),
        ],
        compiler_params=pltpu.CompilerParams(use_tc_tiling_on_sc=False,
                                             needs_layout_passes=False,
                                             disable_bounds_checks=True),
    )
    def body(x_hbm, tok_hbm, pos_hbm, out_hbm,
             idx_v, row_v, off_v, rows_v, wout_v, pos_v, isem, gsem, wsem):
        wid = lax.axis_index("s") * NC + lax.axis_index("c")
        col0 = wid * 128
        pltpu.sync_copy(pos_hbm, pos_v)

        io = lax.iota(jnp.int32, 16)
        rowp = lax.shift_right_logical(io, 3)
        colp = (io & 7) * 128

        def idx_start(g, b):
            t0 = g * G
            for k in range(G):
                pltpu.async_copy(x_hbm.at[pl.ds((t0 + k) * B + col0, 128)],
                                 idx_v.at[b, pl.ds(k * 128, 128)], isem.at[b])

        def idx_wait(b):
            for k in range(G):
                pltpu.make_async_copy(
                    x_hbm.at[pl.ds(col0, 128)],
                    idx_v.at[b, pl.ds(k * 128, 128)], isem.at[b]).wait()

        def idx_transform(b):
            for i in range(GR // 16):
                v = idx_v[b, pl.ds(i * 16, 16)]
                row = (lax.shift_right_logical(v, 9) * 128) | (v & 127)
                off = (lax.shift_right_logical(v, 7) & 3) * 32
                row_v[b, pl.ds(i * 16, 16)] = row
                off_v[b, pl.ds(i * 16, 16)] = off

        def gather_start(b):
            pltpu.async_copy(tok_hbm.at[row_v.at[b]], rows_v.at[b],
                             gsem.at[b])

        def gather_wait(b):
            pltpu.make_async_copy(tok_hbm.at[row_v.at[b]], rows_v.at[b],
                                  gsem.at[b]).wait()

        def write_start(g, b):
            t0 = g * G
            pltpu.async_copy(wout_v.at[b],
                             out_hbm.at[pl.ds(t0 * 4, WROWS), wid],
                             wsem.at[b])

        def write_wait(b):
            pltpu.make_async_copy(wout_v.at[b],
                                  out_hbm.at[pl.ds(0, WROWS), wid],
                                  wsem.at[b]).wait()

        def compute(g, b):
            t0 = g * G
            for k in range(G):
                pv0 = pos_v[t0 + k, pl.ds(0, 16)]
                pv1 = pos_v[t0 + k, pl.ds(16, 16)]
                rv0 = rowp + (k * 4)
                rv1 = rv0 + 2

                @plsc.parallel_loop(0, 128, step=1, unroll=U)
                def _(bl):
                    tp = k * 128 + bl
                    rsplat = jnp.full((16,), tp, jnp.int32)
                    offs = plsc.load_gather(off_v.at[b], [rsplat])
                    c0 = offs + io
                    cv = colp + bl
                    plsc.store_scatter(
                        wout_v.at[b], [rv0, cv],
                        plsc.load_gather(rows_v.at[b], [rsplat, c0]) + pv0)
                    plsc.store_scatter(
                        wout_v.at[b], [rv1, cv],
                        plsc.load_gather(rows_v.at[b], [rsplat, c0 + 16])
                        + pv1)

        idx_start(0, 0)
        idx_start(1, 1)
        idx_wait(0)
        idx_transform(0)
        gather_start(0)

        def ring(i, c):
            for b in (0, 1):
                g = 2 * i + b
                nb = 1 - b

                @pl.when(g + 1 < NG)
                def _():
                    idx_wait(nb)
                    idx_transform(nb)
                    gather_start(nb)

                gather_wait(b)

                @pl.when(g + 2 < NG)
                def _():
                    idx_start(g + 2, b)

                @pl.when(g >= 2)
                def _():
                    write_wait(b)

                compute(g, b)
                write_start(g, b)
            return c

        lax.fori_loop(0, NG // 2, ring, 0)
        write_wait(0)
        write_wait(1)

    return body(xtf, packed, pos_table)


def kernel(x, token_table, pos_table):
    xtf = x.T.reshape(B * T)
    packed = _tc_detile(token_table.T)
    out = _sc_embed(xtf, packed, pos_table)
    return (out.reshape(T, 4, NW, 8, 128).transpose(2, 4, 0, 1, 3)
            .reshape(B, T, D))


# speedup vs baseline: 4.4318x; 1.8823x over previous
import functools

import jax
import jax.numpy as jnp
from jax import lax
from jax.experimental import pallas as pl
from jax.experimental.pallas import tpu as pltpu
from jax.experimental.pallas import tpu_sc as plsc

B = 4096
T = 200
D = 32
V = 1000000
NC = 2
NS = 16
NW = NC * NS
G = 4
NG = T // G
GR = G * 128
WROWS = G * 4
U = 8
TI = 32768
NBLK = (V + TI - 1) // TI
PR = NBLK * (TI // 4)


def _tc_detile(tt):
    def body(in_ref, out_ref):
        tr = jnp.transpose(in_ref[...])
        packs = [
            jnp.concatenate(
                [tr[512 * q + 128 * a:512 * q + 128 * (a + 1), :]
                 for a in range(4)], axis=1)
            for q in range(TI // 512)]
        out_ref[...] = jnp.concatenate(packs, axis=0)

    return pl.pallas_call(
        body,
        grid=(NBLK,),
        in_specs=[pl.BlockSpec((D, TI), lambda i: (0, i))],
        out_specs=pl.BlockSpec((TI // 4, 128), lambda i: (i, 0)),
        out_shape=jax.ShapeDtypeStruct((PR, 128), jnp.float32),
    )(tt)


def _sc_embed(xtf, packed, pos_table):
    mesh = plsc.VectorSubcoreMesh(core_axis_name="c", subcore_axis_name="s")

    @functools.partial(
        pl.kernel,
        mesh=mesh,
        out_type=jax.ShapeDtypeStruct((T * 4, NW, 8, 128), jnp.float32),
        scratch_types=[
            pltpu.VMEM((2, GR), jnp.int32),
            pltpu.VMEM((2, GR), jnp.int32),
            pltpu.VMEM((2, GR, D), jnp.float32),
            pltpu.VMEM((2, WROWS * 8, 129), jnp.float32),
            pltpu.VMEM((T, D), jnp.float32),
            pltpu.SemaphoreType.DMA((2,)),
            pltpu.SemaphoreType.DMA((2,)),
            pltpu.SemaphoreType.DMA((2,)),
        ],
        compiler_params=pltpu.CompilerParams(use_tc_tiling_on_sc=False,
                                             needs_layout_passes=False,
                                             disable_bounds_checks=True),
    )
    def body(x_hbm, tok_hbm, pos_hbm, out_hbm,
             idx_v, row_v, rows_v, wout_v, pos_v, isem, gsem, wsem):
        wid = lax.axis_index("s") * NC + lax.axis_index("c")
        col0 = wid * 128
        pltpu.sync_copy(pos_hbm, pos_v)

        io = lax.iota(jnp.int32, 16)
        rowp = lax.shift_right_logical(io, 3)
        colp = (io & 7) * 128

        def idx_start(g, b):
            t0 = g * G
            for k in range(G):
                pltpu.async_copy(x_hbm.at[pl.ds((t0 + k) * B + col0, 128)],
                                 idx_v.at[b, pl.ds(k * 128, 128)], isem.at[b])

        def idx_wait(b):
            for k in range(G):
                pltpu.make_async_copy(
                    x_hbm.at[pl.ds(col0, 128)],
                    idx_v.at[b, pl.ds(k * 128, 128)], isem.at[b]).wait()

        def idx_transform(b):
            for i in range(GR // 16):
                v = idx_v[b, pl.ds(i * 16, 16)]
                row = ((lax.shift_right_logical(v, 9) * 512)
                       | ((v & 127) * 4)
                       | (lax.shift_right_logical(v, 7) & 3))
                row_v[b, pl.ds(i * 16, 16)] = row

        def gather_start(b):
            pltpu.async_copy(tok_hbm.at[row_v.at[b]], rows_v.at[b],
                             gsem.at[b])

        def gather_wait(b):
            pltpu.make_async_copy(tok_hbm.at[row_v.at[b]], rows_v.at[b],
                                  gsem.at[b]).wait()

        def write_start(g, b):
            t0 = g * G
            for w in range(WROWS):
                pltpu.async_copy(
                    wout_v.at[b, pl.ds(w * 8, 8), pl.ds(0, 128)],
                    out_hbm.at[t0 * 4 + w, wid], wsem.at[b])

        def write_wait(b):
            for w in range(WROWS):
                pltpu.make_async_copy(
                    wout_v.at[b, pl.ds(w * 8, 8), pl.ds(0, 128)],
                    out_hbm.at[0, wid], wsem.at[b]).wait()

        def compute(g, b):
            t0 = g * G
            for k in range(G):
                pv0 = pos_v[t0 + k, pl.ds(0, 16)]
                pv1 = pos_v[t0 + k, pl.ds(16, 16)]
                rv0 = io + (k * 32)
                rv1 = rv0 + 16

                @plsc.parallel_loop(0, 128, step=1, unroll=U)
                def _(bl):
                    cv = jnp.full((16,), bl, jnp.int32)
                    r = k * 128 + bl
                    plsc.store_scatter(wout_v.at[b], [rv0, cv],
                                       rows_v[b, r, pl.ds(0, 16)] + pv0)
                    plsc.store_scatter(wout_v.at[b], [rv1, cv],
                                       rows_v[b, r, pl.ds(16, 16)] + pv1)

        idx_start(0, 0)
        idx_start(1, 1)
        idx_wait(0)
        idx_transform(0)
        gather_start(0)

        def ring(i, c):
            for b in (0, 1):
                g = 2 * i + b
                nb = 1 - b

                @pl.when(g + 1 < NG)
                def _():
                    idx_wait(nb)
                    idx_transform(nb)
                    gather_start(nb)

                gather_wait(b)

                @pl.when(g + 2 < NG)
                def _():
                    idx_start(g + 2, b)

                @pl.when(g >= 2)
                def _():
                    write_wait(b)

                compute(g, b)
                write_start(g, b)
            return c

        lax.fori_loop(0, NG // 2, ring, 0)
        write_wait(0)
        write_wait(1)

    return body(xtf, packed, pos_table)


def kernel(x, token_table, pos_table):
    xtf = x.T.reshape(B * T)
    packed = _tc_detile(token_table.T).reshape(PR * 4, D)
    out = _sc_embed(xtf, packed, pos_table)
    return (out.reshape(T, 4, NW, 8, 128).transpose(2, 4, 0, 1, 3)
            .reshape(B, T, D))
